# baseline (speedup 1.0000x reference)
import functools

import jax
import jax.numpy as jnp
from jax import lax
from jax.experimental import pallas as pl
from jax.experimental.pallas import tpu as pltpu

_RB = 8192


def _mlp_logsoftmax(idx, emb_table, W1, b1, W2T, b2):
    M, H = W2T.shape
    D = emb_table.shape[1]
    RB = _RB
    N = pl.cdiv(M, RB)
    NR = pl.cdiv(N, 8)
    NPAD = NR * 8

    def body(idx_ref, emb_ref, w1_ref, b1_ref, w2t_ref, b2_ref, out_ref,
             out2_ref, out1_ref, m_ref, s_ref, logz_ref):
        i = pl.program_id(0)

        @pl.when(i == 0)
        def _():
            sub = idx_ref[0] % 8
            e = emb_ref[pl.ds(sub, 1), :]
            h = lax.dot_general(e, w1_ref[...], (((1,), (0,)), ((), ())),
                                preferred_element_type=jnp.float32)
            out1_ref[...] = jnp.maximum(h + b1_ref[...], 0.0)
            m_ref[...] = jnp.full((8, 128), -jnp.inf, jnp.float32)
            s_ref[...] = jnp.zeros((8, 128), jnp.float32)
            for rr in range((N - 1) // 8 * 8, NPAD, 8):
                out2_ref[pl.ds(rr, 8), :] = jnp.full((8, RB), -jnp.inf,
                                                     jnp.float32)

        @pl.when(i < N)
        def _():
            x = lax.dot_general(out1_ref[...], w2t_ref[...],
                                (((1,), (1,)), ((), ())),
                                preferred_element_type=jnp.float32)
            x = x + b2_ref[...]
            valid = M - i * RB
            lane = lax.broadcasted_iota(jnp.int32, (1, RB), 1)
            x = jnp.where(lane < valid, x, -jnp.inf)
            out2_ref[pl.ds(i, 1), :] = x

        @pl.when(jnp.logical_and(i >= N, i < N + NR))
        def _():
            j = i - N
            blk = out2_ref[pl.ds(j * 8, 8), :]
            xs = blk.reshape(8, RB // 128, 128)
            bm = jnp.max(xs, axis=1)
            m_old = m_ref[...]
            m_new = jnp.maximum(m_old, bm)
            es = jnp.exp(xs - m_new[:, None, :])
            s_ref[...] = s_ref[...] * jnp.exp(m_old - m_new) + jnp.sum(
                es, axis=1)
            m_ref[...] = m_new

        @pl.when(i >= N + NR)
        def _():
            j = i - (N + NR)

            @pl.when(j == 0)
            def _():
                mv = m_ref[...]
                gm = jnp.max(mv)
                z = jnp.sum(s_ref[...] * jnp.exp(mv - gm))
                logz_ref[0] = gm + jnp.log(z)

            out_ref[...] = out2_ref[pl.ds(j * 8, 8), :] - logz_ref[0]

    grid_spec = pltpu.PrefetchScalarGridSpec(
        num_scalar_prefetch=1,
        grid=(N + 2 * NR,),
        in_specs=[
            pl.BlockSpec((8, D), lambda i, s: (s[0] // 8, 0)),
            pl.BlockSpec(W1.shape, lambda i, s: (0, 0)),
            pl.BlockSpec((1, H), lambda i, s: (0, 0)),
            pl.BlockSpec((_RB, H), lambda i, s: (jnp.minimum(i, N - 1), 0)),
            pl.BlockSpec((1, _RB), lambda i, s: (0, jnp.minimum(i, N - 1))),
        ],
        out_specs=pl.BlockSpec(
            (8, _RB), lambda i, s: (jnp.maximum(i - (N + NR), 0), 0)),
        scratch_shapes=[
            pltpu.VMEM((NPAD, _RB), jnp.float32),
            pltpu.VMEM((1, H), jnp.float32),
            pltpu.VMEM((8, 128), jnp.float32),
            pltpu.VMEM((8, 128), jnp.float32),
            pltpu.SMEM((1,), jnp.float32),
        ],
    )

    out = pl.pallas_call(
        body,
        grid_spec=grid_spec,
        out_shape=jax.ShapeDtypeStruct((NPAD, _RB), jnp.float32),
        compiler_params=pltpu.CompilerParams(
            dimension_semantics=("arbitrary",),
        ),
    )(idx, emb_table, W1, b1.reshape(1, H), W2T, b2.reshape(1, M))
    return out


def kernel(inputs, emb_table, W1, b1, W2, b2):
    idx = inputs.astype(jnp.int32)
    out = _mlp_logsoftmax(idx, emb_table, W1, b1, W2.T, b2)
    M = W2.shape[1]
    return out.reshape(-1)[:M].reshape(3, -1)
